# baseline (speedup 1.0000x reference)
import functools

import jax
import jax.numpy as jnp
from jax.experimental import pallas as pl
from jax.experimental.pallas import tpu as pltpu
from jax.experimental.pallas import tpu_sc as plsc

_N = 1_000_000
_D = 64
_K = 8
_TAU = 0.07
_ALPHA = 0.8
_S = 8
_LC = 7808
_G = 16
_REG = _G * _LC
_MAIN = _S * _REG
_T = _N - _MAIN
_NEG = -1e30
_BIG = 2**31 - 1


def _scan_body(*refs):
    blk_refs = refs[:_S]
    tail_ref, q_ref, vals_ref, idx_ref = refs[_S], refs[_S + 1], refs[_S + 2], refs[_S + 3]
    b = pl.program_id(0)
    qv = q_ref[...]
    parts = []
    for blk_ref in blk_refs:
        p = blk_ref[...] * qv
        parts.append(jnp.sum(p, axis=0)[None, :])
    s8 = jnp.concatenate(parts, axis=0)
    st = jnp.sum(tail_ref[...] * qv, axis=0)[None, :]
    st = jnp.where(b == 0, st, _NEG)
    r = jax.lax.broadcasted_iota(jnp.int32, s8.shape, 0)
    c = jax.lax.broadcasted_iota(jnp.int32, s8.shape, 1)
    g8 = r * _REG + b * _LC + c
    gt = _MAIN + jax.lax.broadcasted_iota(jnp.int32, st.shape, 1)
    lane = jax.lax.broadcasted_iota(jnp.int32, (1, 128), 1)
    vvec = jnp.full((1, 128), _NEG, jnp.float32)
    ivec = jnp.zeros((1, 128), jnp.int32)
    for k in range(_K):
        m = jnp.maximum(jnp.max(s8), jnp.max(st))
        fk = jnp.minimum(
            jnp.min(jnp.where(s8 == m, g8, _BIG)),
            jnp.min(jnp.where(st == m, gt, _BIG)))
        vvec = jnp.where(lane == k, m, vvec)
        ivec = jnp.where(lane == k, fk, ivec)
        s8 = jnp.where(g8 == fk, _NEG, s8)
        st = jnp.where(gt == fk, _NEG, st)
    vals_ref[...] = vvec.reshape(1, 1, 128)
    idx_ref[...] = ivec.reshape(1, 1, 128)


def _combine_body(vals_ref, idx_ref, rc_ref, bank_ref, out_ref, cols_v, sem):
    s = vals_ref[...].reshape(_G, 128)
    gi = idx_ref[...].reshape(_G, 128)
    tv, ti = [], []
    for k in range(_K):
        m = jnp.max(s)
        fk = jnp.min(jnp.where(s == m, gi, _BIG))
        tv.append(m)
        ti.append(fk)
        s = jnp.where(gi == fk, _NEG, s)
    cps = [
        pltpu.make_async_copy(
            bank_ref.at[:, pl.ds((ti[k] // 128) * 128, 128)],
            cols_v.at[:, pl.ds(k * 128, 128)], sem)
        for k in range(_K)
    ]
    for cp in cps:
        cp.start()
    for cp in cps:
        cp.wait()
    m0 = tv[0]
    ws = [jnp.exp((tv[k] - m0) / _TAU) for k in range(_K)]
    den = ws[0]
    for k in range(1, _K):
        den = den + ws[k]
    lane64 = jax.lax.broadcasted_iota(jnp.int32, (_D, 128), 1)
    ret = jnp.zeros((_D, 1), jnp.float32)
    for k in range(_K):
        win = cols_v[:, k * 128:(k + 1) * 128]
        col = jnp.sum(
            jnp.where(lane64 == ti[k] % 128, win, 0.0), axis=1, keepdims=True)
        ret = ret + (ws[k] / den) * col
    nrm = jnp.sqrt(jnp.sum(ret * ret))
    retn = ret / jnp.maximum(nrm, 1e-12)
    rc = rc_ref[...]
    scale = jnp.maximum(jnp.sqrt(jnp.sum(rc * rc)), 1e-6)
    out_ref[...] = _ALPHA * rc + (1.0 - _ALPHA) * retn * scale


_NCAND = _G * 128


def _dyng(x, idx):
    return jax.lax.gather(
        x, idx[:, None],
        jax.lax.GatherDimensionNumbers(
            offset_dims=(), collapsed_slice_dims=(0,), start_index_map=(0,)),
        (1,), mode=jax.lax.GatherScatterMode.PROMISE_IN_BOUNDS)


def _mk_butterfly(lane, op):
    def f(x):
        for sh in (8, 4, 2, 1):
            x = op(x, _dyng(x, jax.lax.bitwise_xor(lane, sh)))
        return x
    return f


def _sc_combine_body(vals_hbm, idx_hbm, rc_hbm, bank_hbm, out_hbm,
                     vals_v, idx_v, win_v, rc_v, out_v):
    wid = jax.lax.axis_index("s") * 2 + jax.lax.axis_index("c")
    pltpu.sync_copy(vals_hbm, vals_v)
    pltpu.sync_copy(idx_hbm, idx_v)
    pltpu.sync_copy(rc_hbm, rc_v)
    vregs = [vals_v[pl.ds(r * 16, 16)] for r in range(_NCAND // 16)]
    gregs = [idx_v[pl.ds(r * 16, 16)] for r in range(_NCAND // 16)]
    lane = jax.lax.broadcasted_iota(jnp.int32, (16,), 0)
    allmax = _mk_butterfly(lane, jnp.maximum)
    allmin = _mk_butterfly(lane, jnp.minimum)
    allsum = _mk_butterfly(lane, jnp.add)
    zeros16 = jnp.zeros((16,), jnp.int32)
    tvec = jnp.full((16,), _NEG, jnp.float32)
    ivec = zeros16
    for k in range(_K):
        acc = vregs[0]
        for r in range(1, len(vregs)):
            acc = jnp.maximum(acc, vregs[r])
        m = allmax(acc)
        cacc = jnp.where(vregs[0] == m, gregs[0], _BIG)
        for r in range(1, len(vregs)):
            cacc = jnp.minimum(
                cacc, jnp.where(vregs[r] == m, gregs[r], _BIG))
        fk = allmin(cacc)
        tvec = jnp.where(lane == k, m, tvec)
        ivec = jnp.where(lane == k, fk, ivec)
        vregs = [jnp.where(gregs[r] == fk, _NEG, vregs[r])
                 for r in range(len(vregs))]
    tv0 = _dyng(tvec, zeros16)
    w = jnp.exp((tvec - tv0) / _TAU)
    wn = w / allsum(w)
    offs = jax.lax.bitwise_and(ivec, 127)
    offmod = jax.lax.bitwise_and(ivec, 15)
    acc4 = [jnp.zeros((16,), jnp.float32) for _ in range(4)]
    for k in range(_K):
        idx_k = ivec[k]
        start = jax.lax.shift_right_logical(idx_k, 7) * 128
        seg = jax.lax.shift_right_logical(
            jax.lax.bitwise_and(idx_k, 127), 4) * 16
        pltpu.sync_copy(bank_hbm.at[:, pl.ds(start, 128)], win_v)
        wk = _dyng(wn, zeros16 + k)
        omv = _dyng(offmod, zeros16 + k)
        for c4 in range(4):
            col = jnp.zeros((16,), jnp.float32)
            for i in range(16):
                row = win_v.at[c4 * 16 + i]
                sv = row[pl.ds(seg, 16)]
                val = _dyng(sv, omv)
                col = jnp.where(lane == i, val, col)
            acc4[c4] = acc4[c4] + wk * col
    def bab_sqrt(x):
        s = 0.5 * (1.0 + x)
        for _ in range(40):
            s = 0.5 * (s + x / s)
        return s

    ssv = allsum(acc4[0] * acc4[0])
    for c4 in range(1, 4):
        ssv = ssv + allsum(acc4[c4] * acc4[c4])
    nrm = jnp.maximum(bab_sqrt(ssv), 1e-12)
    rc4 = [rc_v[pl.ds(c4 * 16, 16)] for c4 in range(4)]
    rssv = allsum(rc4[0] * rc4[0])
    for c4 in range(1, 4):
        rssv = rssv + allsum(rc4[c4] * rc4[c4])
    scale = jnp.maximum(bab_sqrt(rssv), 1e-6)
    for c4 in range(4):
        retn = acc4[c4] / nrm
        out_v[pl.ds(c4 * 16, 16)] = (
            _ALPHA * rc4[c4] + (1.0 - _ALPHA) * retn * scale)

    @pl.when(wid == 0)
    def _():
        pltpu.sync_copy(out_v, out_hbm)


def _sc_combine(vals, idx, recent, bank_t):
    mesh = plsc.VectorSubcoreMesh(
        core_axis_name="c", subcore_axis_name="s", num_cores=2,
        num_subcores=16)
    kern = functools.partial(
        pl.kernel,
        out_type=jax.ShapeDtypeStruct((_D,), jnp.float32),
        mesh=mesh,
        scratch_types=[
            pltpu.VMEM((_NCAND,), jnp.float32),
            pltpu.VMEM((_NCAND,), jnp.int32),
            pltpu.VMEM((_D, 128), jnp.float32),
            pltpu.VMEM((_D,), jnp.float32),
            pltpu.VMEM((_D,), jnp.float32),
        ],
    )(_sc_combine_body)
    return kern(vals, idx, recent, bank_t)


def _mk_stream_spec(si):
    return pl.BlockSpec((_D, _LC), lambda b, _s=si: (0, _s * _G + b))


def kernel(mem_bank, query, recent_concept):
    bank_t = mem_bank.T
    tail = jax.lax.slice(bank_t, (0, _MAIN), (_D, _N))
    vals, idx = pl.pallas_call(
        _scan_body,
        grid=(_G,),
        in_specs=[_mk_stream_spec(si) for si in range(_S)] + [
            pl.BlockSpec((_D, _T), lambda b: (0, 0)),
            pl.BlockSpec((_D, 1), lambda b: (0, 0)),
        ],
        out_specs=[
            pl.BlockSpec((1, 1, 128), lambda b: (b, 0, 0)),
            pl.BlockSpec((1, 1, 128), lambda b: (b, 0, 0)),
        ],
        out_shape=[
            jax.ShapeDtypeStruct((_G, 1, 128), jnp.float32),
            jax.ShapeDtypeStruct((_G, 1, 128), jnp.int32),
        ],
    )(*([bank_t] * _S), tail, query.reshape(_D, 1))
    return _sc_combine(
        vals.reshape(_NCAND), idx.reshape(_NCAND), recent_concept, bank_t)


# speedup vs baseline: 1.0291x; 1.0291x over previous
import functools

import jax
import jax.numpy as jnp
from jax.experimental import pallas as pl
from jax.experimental.pallas import tpu as pltpu
from jax.experimental.pallas import tpu_sc as plsc

_N = 1_000_000
_D = 64
_K = 8
_TAU = 0.07
_ALPHA = 0.8
_S = 8
_LC = 7808
_G = 16
_REG = _G * _LC
_MAIN = _S * _REG
_T = _N - _MAIN
_NEG = -1e30
_BIG = 2**31 - 1


def _scan_body(*refs):
    blk_refs = refs[:_S]
    tail_ref, q_ref, vals_ref, idx_ref = refs[_S], refs[_S + 1], refs[_S + 2], refs[_S + 3]
    b = pl.program_id(0)
    qv = q_ref[...]
    parts = []
    for blk_ref in blk_refs:
        p = blk_ref[...] * qv
        parts.append(jnp.sum(p, axis=0)[None, :])
    s8 = jnp.concatenate(parts, axis=0)
    st = jnp.sum(tail_ref[...] * qv, axis=0)[None, :]
    st = jnp.where(b == 0, st, _NEG)
    r = jax.lax.broadcasted_iota(jnp.int32, s8.shape, 0)
    c = jax.lax.broadcasted_iota(jnp.int32, s8.shape, 1)
    g8 = r * _REG + b * _LC + c
    gt = _MAIN + jax.lax.broadcasted_iota(jnp.int32, st.shape, 1)
    lane = jax.lax.broadcasted_iota(jnp.int32, (1, 128), 1)
    vvec = jnp.full((1, 128), _NEG, jnp.float32)
    ivec = jnp.zeros((1, 128), jnp.int32)
    for k in range(_K):
        m = jnp.maximum(jnp.max(s8), jnp.max(st))
        fk = jnp.minimum(
            jnp.min(jnp.where(s8 == m, g8, _BIG)),
            jnp.min(jnp.where(st == m, gt, _BIG)))
        vvec = jnp.where(lane == k, m, vvec)
        ivec = jnp.where(lane == k, fk, ivec)
        s8 = jnp.where(g8 == fk, _NEG, s8)
        st = jnp.where(gt == fk, _NEG, st)
    vals_ref[...] = vvec.reshape(1, 1, 128)
    idx_ref[...] = ivec.reshape(1, 1, 128)


def _combine_body(vals_ref, idx_ref, rc_ref, bank_ref, out_ref, cols_v, sem):
    s = vals_ref[...].reshape(_G, 128)
    gi = idx_ref[...].reshape(_G, 128)
    tv, ti = [], []
    for k in range(_K):
        m = jnp.max(s)
        fk = jnp.min(jnp.where(s == m, gi, _BIG))
        tv.append(m)
        ti.append(fk)
        s = jnp.where(gi == fk, _NEG, s)
    cps = [
        pltpu.make_async_copy(
            bank_ref.at[:, pl.ds((ti[k] // 128) * 128, 128)],
            cols_v.at[:, pl.ds(k * 128, 128)], sem)
        for k in range(_K)
    ]
    for cp in cps:
        cp.start()
    for cp in cps:
        cp.wait()
    m0 = tv[0]
    ws = [jnp.exp((tv[k] - m0) / _TAU) for k in range(_K)]
    den = ws[0]
    for k in range(1, _K):
        den = den + ws[k]
    lane64 = jax.lax.broadcasted_iota(jnp.int32, (_D, 128), 1)
    ret = jnp.zeros((_D, 1), jnp.float32)
    for k in range(_K):
        win = cols_v[:, k * 128:(k + 1) * 128]
        col = jnp.sum(
            jnp.where(lane64 == ti[k] % 128, win, 0.0), axis=1, keepdims=True)
        ret = ret + (ws[k] / den) * col
    nrm = jnp.sqrt(jnp.sum(ret * ret))
    retn = ret / jnp.maximum(nrm, 1e-12)
    rc = rc_ref[...]
    scale = jnp.maximum(jnp.sqrt(jnp.sum(rc * rc)), 1e-6)
    out_ref[...] = _ALPHA * rc + (1.0 - _ALPHA) * retn * scale


_NCAND = _G * 128


def _dyng(x, idx):
    return jax.lax.gather(
        x, idx[:, None],
        jax.lax.GatherDimensionNumbers(
            offset_dims=(), collapsed_slice_dims=(0,), start_index_map=(0,)),
        (1,), mode=jax.lax.GatherScatterMode.PROMISE_IN_BOUNDS)


def _mk_butterfly(lane, op):
    def f(x):
        for sh in (8, 4, 2, 1):
            x = op(x, _dyng(x, jax.lax.bitwise_xor(lane, sh)))
        return x
    return f


def _sc_combine_body(vals_hbm, idx_hbm, rc_hbm, bank_hbm, out_hbm,
                     vals_v, idx_v, win_v, rc_v, out_v, sem):
    wid = jax.lax.axis_index("s") * 2 + jax.lax.axis_index("c")
    pltpu.sync_copy(vals_hbm, vals_v)
    pltpu.sync_copy(idx_hbm, idx_v)
    pltpu.sync_copy(rc_hbm, rc_v)
    vregs = [vals_v[pl.ds(r * 16, 16)] for r in range(_NCAND // 16)]
    gregs = [idx_v[pl.ds(r * 16, 16)] for r in range(_NCAND // 16)]
    lane = jax.lax.broadcasted_iota(jnp.int32, (16,), 0)
    allmax = _mk_butterfly(lane, jnp.maximum)
    allmin = _mk_butterfly(lane, jnp.minimum)
    allsum = _mk_butterfly(lane, jnp.add)
    zeros16 = jnp.zeros((16,), jnp.int32)
    tvec = jnp.full((16,), _NEG, jnp.float32)
    ivec = zeros16
    for k in range(_K):
        acc = vregs[0]
        for r in range(1, len(vregs)):
            acc = jnp.maximum(acc, vregs[r])
        m = allmax(acc)
        cacc = jnp.where(vregs[0] == m, gregs[0], _BIG)
        for r in range(1, len(vregs)):
            cacc = jnp.minimum(
                cacc, jnp.where(vregs[r] == m, gregs[r], _BIG))
        fk = allmin(cacc)
        tvec = jnp.where(lane == k, m, tvec)
        ivec = jnp.where(lane == k, fk, ivec)
        vregs = [jnp.where(gregs[r] == fk, _NEG, vregs[r])
                 for r in range(len(vregs))]
    tv0 = _dyng(tvec, zeros16)
    w = jnp.exp((tvec - tv0) / _TAU)
    wn = w / allsum(w)
    offmod = jax.lax.bitwise_and(ivec, 15)
    cps = []
    for k in range(_K):
        idx_k = ivec[k]
        start = jax.lax.shift_right_logical(idx_k, 7) * 128
        cps.append(pltpu.make_async_copy(
            bank_hbm.at[:, pl.ds(start, 128)], win_v.at[k], sem))
    for cp in cps:
        cp.start()
    for cp in cps:
        cp.wait()
    acc4 = [jnp.zeros((16,), jnp.float32) for _ in range(4)]
    for k in range(_K):
        idx_k = ivec[k]
        seg = jax.lax.shift_right_logical(
            jax.lax.bitwise_and(idx_k, 127), 4) * 16
        wk = _dyng(wn, zeros16 + k)
        omv = _dyng(offmod, zeros16 + k)
        for c4 in range(4):
            col = jnp.zeros((16,), jnp.float32)
            for i in range(16):
                row = win_v.at[k, c4 * 16 + i]
                sv = row[pl.ds(seg, 16)]
                val = _dyng(sv, omv)
                col = jnp.where(lane == i, val, col)
            acc4[c4] = acc4[c4] + wk * col
    def bab_sqrt(x):
        s = 0.5 * (1.0 + x)
        for _ in range(40):
            s = 0.5 * (s + x / s)
        return s

    ssv = allsum(acc4[0] * acc4[0])
    for c4 in range(1, 4):
        ssv = ssv + allsum(acc4[c4] * acc4[c4])
    nrm = jnp.maximum(bab_sqrt(ssv), 1e-12)
    rc4 = [rc_v[pl.ds(c4 * 16, 16)] for c4 in range(4)]
    rssv = allsum(rc4[0] * rc4[0])
    for c4 in range(1, 4):
        rssv = rssv + allsum(rc4[c4] * rc4[c4])
    scale = jnp.maximum(bab_sqrt(rssv), 1e-6)
    for c4 in range(4):
        retn = acc4[c4] / nrm
        out_v[pl.ds(c4 * 16, 16)] = (
            _ALPHA * rc4[c4] + (1.0 - _ALPHA) * retn * scale)

    @pl.when(wid == 0)
    def _():
        pltpu.sync_copy(out_v, out_hbm)


def _sc_combine(vals, idx, recent, bank_t):
    mesh = plsc.VectorSubcoreMesh(
        core_axis_name="c", subcore_axis_name="s", num_cores=2,
        num_subcores=16)
    kern = functools.partial(
        pl.kernel,
        out_type=jax.ShapeDtypeStruct((_D,), jnp.float32),
        mesh=mesh,
        scratch_types=[
            pltpu.VMEM((_NCAND,), jnp.float32),
            pltpu.VMEM((_NCAND,), jnp.int32),
            pltpu.VMEM((_K, _D, 128), jnp.float32),
            pltpu.VMEM((_D,), jnp.float32),
            pltpu.VMEM((_D,), jnp.float32),
            pltpu.SemaphoreType.DMA,
        ],
    )(_sc_combine_body)
    return kern(vals, idx, recent, bank_t)


def _mk_stream_spec(si):
    return pl.BlockSpec((_D, _LC), lambda b, _s=si: (0, _s * _G + b))


def kernel(mem_bank, query, recent_concept):
    bank_t = mem_bank.T
    tail = jax.lax.slice(bank_t, (0, _MAIN), (_D, _N))
    vals, idx = pl.pallas_call(
        _scan_body,
        grid=(_G,),
        in_specs=[_mk_stream_spec(si) for si in range(_S)] + [
            pl.BlockSpec((_D, _T), lambda b: (0, 0)),
            pl.BlockSpec((_D, 1), lambda b: (0, 0)),
        ],
        out_specs=[
            pl.BlockSpec((1, 1, 128), lambda b: (b, 0, 0)),
            pl.BlockSpec((1, 1, 128), lambda b: (b, 0, 0)),
        ],
        out_shape=[
            jax.ShapeDtypeStruct((_G, 1, 128), jnp.float32),
            jax.ShapeDtypeStruct((_G, 1, 128), jnp.int32),
        ],
    )(*([bank_t] * _S), tail, query.reshape(_D, 1))
    return _sc_combine(
        vals.reshape(_NCAND), idx.reshape(_NCAND), recent_concept, bank_t)


# speedup vs baseline: 1.0406x; 1.0111x over previous
import functools

import jax
import jax.numpy as jnp
from jax.experimental import pallas as pl
from jax.experimental.pallas import tpu as pltpu
from jax.experimental.pallas import tpu_sc as plsc

_N = 1_000_000
_D = 64
_K = 8
_TAU = 0.07
_ALPHA = 0.8
_S = 8
_LC = 7808
_G = 16
_REG = _G * _LC
_MAIN = _S * _REG
_T = _N - _MAIN
_NEG = -1e30
_BIG = 2**31 - 1


def _scan_body(*refs):
    blk_refs = refs[:_S]
    tail_ref, q_ref, vals_ref, idx_ref = refs[_S], refs[_S + 1], refs[_S + 2], refs[_S + 3]
    b = pl.program_id(0)
    qv = q_ref[...]
    parts = []
    for blk_ref in blk_refs:
        p = blk_ref[...] * qv
        parts.append(jnp.sum(p, axis=0)[None, :])
    s8 = jnp.concatenate(parts, axis=0)
    st = jnp.sum(tail_ref[...] * qv, axis=0)[None, :]
    st = jnp.where(b == 0, st, _NEG)
    r = jax.lax.broadcasted_iota(jnp.int32, s8.shape, 0)
    c = jax.lax.broadcasted_iota(jnp.int32, s8.shape, 1)
    g8 = r * _REG + b * _LC + c
    gt = _MAIN + jax.lax.broadcasted_iota(jnp.int32, st.shape, 1)
    lane = jax.lax.broadcasted_iota(jnp.int32, (1, 128), 1)
    vvec = jnp.full((1, 128), _NEG, jnp.float32)
    ivec = jnp.zeros((1, 128), jnp.int32)
    for k in range(_K):
        m = jnp.maximum(jnp.max(s8), jnp.max(st))
        fk = jnp.minimum(
            jnp.min(jnp.where(s8 == m, g8, _BIG)),
            jnp.min(jnp.where(st == m, gt, _BIG)))
        vvec = jnp.where(lane == k, m, vvec)
        ivec = jnp.where(lane == k, fk, ivec)
        s8 = jnp.where(g8 == fk, _NEG, s8)
        st = jnp.where(gt == fk, _NEG, st)
    vals_ref[...] = vvec.reshape(1, 1, 128)
    idx_ref[...] = ivec.reshape(1, 1, 128)


def _combine_body(vals_ref, idx_ref, rc_ref, bank_ref, out_ref, cols_v, sem):
    s = vals_ref[...].reshape(_G, 128)
    gi = idx_ref[...].reshape(_G, 128)
    tv, ti = [], []
    for k in range(_K):
        m = jnp.max(s)
        fk = jnp.min(jnp.where(s == m, gi, _BIG))
        tv.append(m)
        ti.append(fk)
        s = jnp.where(gi == fk, _NEG, s)
    cps = [
        pltpu.make_async_copy(
            bank_ref.at[:, pl.ds((ti[k] // 128) * 128, 128)],
            cols_v.at[:, pl.ds(k * 128, 128)], sem)
        for k in range(_K)
    ]
    for cp in cps:
        cp.start()
    for cp in cps:
        cp.wait()
    m0 = tv[0]
    ws = [jnp.exp((tv[k] - m0) / _TAU) for k in range(_K)]
    den = ws[0]
    for k in range(1, _K):
        den = den + ws[k]
    lane64 = jax.lax.broadcasted_iota(jnp.int32, (_D, 128), 1)
    ret = jnp.zeros((_D, 1), jnp.float32)
    for k in range(_K):
        win = cols_v[:, k * 128:(k + 1) * 128]
        col = jnp.sum(
            jnp.where(lane64 == ti[k] % 128, win, 0.0), axis=1, keepdims=True)
        ret = ret + (ws[k] / den) * col
    nrm = jnp.sqrt(jnp.sum(ret * ret))
    retn = ret / jnp.maximum(nrm, 1e-12)
    rc = rc_ref[...]
    scale = jnp.maximum(jnp.sqrt(jnp.sum(rc * rc)), 1e-6)
    out_ref[...] = _ALPHA * rc + (1.0 - _ALPHA) * retn * scale


_NCAND = _G * 128


def _dyng(x, idx):
    return jax.lax.gather(
        x, idx[:, None],
        jax.lax.GatherDimensionNumbers(
            offset_dims=(), collapsed_slice_dims=(0,), start_index_map=(0,)),
        (1,), mode=jax.lax.GatherScatterMode.PROMISE_IN_BOUNDS)


def _mk_butterfly(lane, op):
    def f(x):
        for sh in (8, 4, 2, 1):
            x = op(x, _dyng(x, jax.lax.bitwise_xor(lane, sh)))
        return x
    return f


def _fsqrt(x):
    i = jax.lax.bitcast_convert_type(x, jnp.int32)
    y = jax.lax.bitcast_convert_type(
        0x5F3759DF - jax.lax.shift_right_logical(i, 1), jnp.float32)
    for _ in range(4):
        y = y * (1.5 - 0.5 * x * y * y)
    return x * y


def _sc_combine_body(vals_hbm, idx_hbm, rc_hbm, bank_hbm, out_hbm,
                     vals_v, idx_v, win_v, rc_v, out_v, sem):
    wid = jax.lax.axis_index("s") * 2 + jax.lax.axis_index("c")
    pltpu.sync_copy(vals_hbm, vals_v)
    pltpu.sync_copy(idx_hbm, idx_v)
    pltpu.sync_copy(rc_hbm, rc_v)
    vregs = [vals_v[pl.ds(r * 16, 16)] for r in range(_NCAND // 16)]
    gregs = [idx_v[pl.ds(r * 16, 16)] for r in range(_NCAND // 16)]
    lane = jax.lax.broadcasted_iota(jnp.int32, (16,), 0)
    allmax = _mk_butterfly(lane, jnp.maximum)
    allmin = _mk_butterfly(lane, jnp.minimum)
    allsum = _mk_butterfly(lane, jnp.add)
    zeros16 = jnp.zeros((16,), jnp.int32)
    tvec = jnp.full((16,), _NEG, jnp.float32)
    ivec = zeros16
    for k in range(_K):
        acc = vregs[0]
        for r in range(1, len(vregs)):
            acc = jnp.maximum(acc, vregs[r])
        m = allmax(acc)
        cacc = jnp.where(vregs[0] == m, gregs[0], _BIG)
        for r in range(1, len(vregs)):
            cacc = jnp.minimum(
                cacc, jnp.where(vregs[r] == m, gregs[r], _BIG))
        fk = allmin(cacc)
        tvec = jnp.where(lane == k, m, tvec)
        ivec = jnp.where(lane == k, fk, ivec)
        vregs = [jnp.where(gregs[r] == fk, _NEG, vregs[r])
                 for r in range(len(vregs))]
    cps = []
    for k in range(_K):
        idx_k = ivec[k]
        start = jax.lax.shift_right_logical(idx_k, 7) * 128
        cps.append(pltpu.make_async_copy(
            bank_hbm.at[:, pl.ds(start, 128)], win_v.at[k], sem))
    for cp in cps:
        cp.start()
    tv0 = _dyng(tvec, zeros16)
    w = jnp.exp((tvec - tv0) / _TAU)
    wn = w / allsum(w)
    offmod = jax.lax.bitwise_and(ivec, 15)
    for cp in cps:
        cp.wait()
    acc4 = [jnp.zeros((16,), jnp.float32) for _ in range(4)]
    for k in range(_K):
        idx_k = ivec[k]
        seg = jax.lax.shift_right_logical(
            jax.lax.bitwise_and(idx_k, 127), 4) * 16
        wk = _dyng(wn, zeros16 + k)
        omv = _dyng(offmod, zeros16 + k)
        for c4 in range(4):
            col = jnp.zeros((16,), jnp.float32)
            for i in range(16):
                row = win_v.at[k, c4 * 16 + i]
                sv = row[pl.ds(seg, 16)]
                val = _dyng(sv, omv)
                col = jnp.where(lane == i, val, col)
            acc4[c4] = acc4[c4] + wk * col
    ssv = allsum(acc4[0] * acc4[0])
    for c4 in range(1, 4):
        ssv = ssv + allsum(acc4[c4] * acc4[c4])
    nrm = jnp.maximum(_fsqrt(ssv), 1e-12)
    rc4 = [rc_v[pl.ds(c4 * 16, 16)] for c4 in range(4)]
    rssv = allsum(rc4[0] * rc4[0])
    for c4 in range(1, 4):
        rssv = rssv + allsum(rc4[c4] * rc4[c4])
    scale = jnp.maximum(_fsqrt(rssv), 1e-6)
    for c4 in range(4):
        retn = acc4[c4] / nrm
        out_v[pl.ds(c4 * 16, 16)] = (
            _ALPHA * rc4[c4] + (1.0 - _ALPHA) * retn * scale)

    @pl.when(wid == 0)
    def _():
        pltpu.sync_copy(out_v, out_hbm)


def _sc_combine(vals, idx, recent, bank_t):
    mesh = plsc.VectorSubcoreMesh(
        core_axis_name="c", subcore_axis_name="s", num_cores=2,
        num_subcores=16)
    kern = functools.partial(
        pl.kernel,
        out_type=jax.ShapeDtypeStruct((_D,), jnp.float32),
        mesh=mesh,
        scratch_types=[
            pltpu.VMEM((_NCAND,), jnp.float32),
            pltpu.VMEM((_NCAND,), jnp.int32),
            pltpu.VMEM((_K, _D, 128), jnp.float32),
            pltpu.VMEM((_D,), jnp.float32),
            pltpu.VMEM((_D,), jnp.float32),
            pltpu.SemaphoreType.DMA,
        ],
    )(_sc_combine_body)
    return kern(vals, idx, recent, bank_t)


def _mk_stream_spec(si):
    return pl.BlockSpec((_D, _LC), lambda b, _s=si: (0, _s * _G + b))


def kernel(mem_bank, query, recent_concept):
    bank_t = mem_bank.T
    tail = jax.lax.slice(bank_t, (0, _MAIN), (_D, _N))
    vals, idx = pl.pallas_call(
        _scan_body,
        grid=(_G,),
        in_specs=[_mk_stream_spec(si) for si in range(_S)] + [
            pl.BlockSpec((_D, _T), lambda b: (0, 0)),
            pl.BlockSpec((_D, 1), lambda b: (0, 0)),
        ],
        out_specs=[
            pl.BlockSpec((1, 1, 128), lambda b: (b, 0, 0)),
            pl.BlockSpec((1, 1, 128), lambda b: (b, 0, 0)),
        ],
        out_shape=[
            jax.ShapeDtypeStruct((_G, 1, 128), jnp.float32),
            jax.ShapeDtypeStruct((_G, 1, 128), jnp.int32),
        ],
    )(*([bank_t] * _S), tail, query.reshape(_D, 1))
    return _sc_combine(
        vals.reshape(_NCAND), idx.reshape(_NCAND), recent_concept, bank_t)
